# baseline (speedup 1.0000x reference)
import functools
import math

import jax
import jax.numpy as jnp
import numpy as np
from jax.experimental import pallas as pl
from jax.experimental.pallas import tpu as pltpu

_N_SPIXELS = 256
_N_ITERS = 5
_NEG = -1e16


def _cells_layout(h, w, n_spixels):
    nw = int(math.sqrt(n_spixels * w / h) + 0.5)
    nh = int(math.sqrt(n_spixels * h / w) + 0.5)
    cw = int(math.ceil(w / nw))
    ch = int(math.ceil(h / nh))
    return nh, nw, ch, cw


def _consts(h, w, nh, nw, ch, cw, b, c):
    j_of_l = np.minimum(np.arange(w) // cw, nw - 1)
    E = np.zeros((3, w, nw), np.float32)
    cbias = np.zeros((3, 1, w), np.float32)
    for t, dx in enumerate((-1, 0, 1)):
        jj = j_of_l + dx
        ok = (jj >= 0) & (jj < nw)
        jc = np.clip(jj, 0, nw - 1)
        E[t, np.arange(w), jc] = 1.0
        cbias[t, 0] = np.where(ok, 0.0, -1e16).astype(np.float32)
    G = np.ascontiguousarray(np.transpose(E, (0, 2, 1)))
    RGbc = np.zeros((b * c, b * c * ch), np.float32)
    for g in range(b * c):
        RGbc[g, g * ch:(g + 1) * ch] = 1.0
    RGb = np.zeros((b, b * ch), np.float32)
    for g in range(b):
        RGb[g, g * ch:(g + 1) * ch] = 1.0
    R = np.zeros((b, b * c), np.float32)
    for bi in range(b):
        R[bi, bi * c:(bi + 1) * c] = 1.0
    return (jnp.asarray(E), jnp.asarray(G), jnp.asarray(cbias),
            jnp.asarray(RGbc), jnp.asarray(RGb), jnp.asarray(R))


def _mm(a, bmat):
    return jax.lax.dot_general(a, bmat, (((1,), (0,)), ((), ())),
                               preferred_element_type=jnp.float32)


def _ssn_body(x_ref, e_ref, g_ref, cbias_ref, rgbc_ref, rgb_ref, rrep_ref,
              q_ref, spfp_ref, spf_s, num_s, den_s,
              *, nh, nw, ch, b, c, n_iters):
    i = pl.program_id(0)
    r = pl.program_id(1)
    w = x_ref.shape[-1]
    bc = b * c
    X = x_ref[...]
    X20 = X.reshape(bc, ch, w)

    @pl.when(jnp.logical_and(i == 0, r == 0))
    def _zero():
        num_s[...] = jnp.zeros_like(num_s)
        den_s[...] = jnp.zeros_like(den_s)

    @pl.when(i == 0)
    def _init():
        colsum = _mm(rgbc_ref[...], X.reshape(bc * ch, w))
        contrib = _mm(colsum, e_ref[1])
        cnt = _mm(_mm(rgb_ref[...], jnp.ones((b * ch, w), jnp.float32)),
                  e_ref[1])
        num_s[pl.ds(r * nw, nw), :] += contrib.T
        den_s[pl.ds(r * nw, nw), :] += cnt.T

    @pl.when(i > 0)
    def _iterate():
        xsq = jnp.sum(X * X, axis=1)
        nd = []
        for dy in (-1, 0, 1):
            rn = r + dy
            rbias = jnp.where(jnp.logical_and(rn >= 0, rn < nh),
                              jnp.float32(0.0), jnp.float32(_NEG))
            rp = jnp.clip(rn, 0, nh - 1)
            S = spf_s[pl.ds(rp * nw, nw), :]
            for t_dx in range(3):
                Map = jax.lax.dot_general(
                    S, g_ref[t_dx], (((0,), (0,)), ((), ())),
                    preferred_element_type=jnp.float32)
                Mb = Map.reshape(b, c, 1, w)
                cross = jnp.sum(X * Mb, axis=1)
                ssq = jnp.sum(Map.reshape(b, c, w) ** 2, axis=1)
                bias = cbias_ref[t_dx] + rbias
                d = (xsq - 2.0 * cross) + ssq[:, None, :]
                nd.append(bias - d)
        m = nd[0]
        for t in range(1, 9):
            m = jnp.maximum(m, nd[t])
        ex = [jnp.exp(nd[t] - m) for t in range(9)]
        s = ex[0]
        for t in range(1, 9):
            s = s + ex[t]
        rs = 1.0 / s
        Xn = X * rs[:, None]

        @pl.when(i == n_iters)
        def _emit_q():
            q_ref[...] = jnp.stack([ex[t] * rs for t in range(9)], axis=1)

        cn_cols = []
        cd_cols = []
        for t_dy in range(3):
            cn = jnp.zeros((bc, nw), jnp.float32)
            cd = jnp.zeros((b, nw), jnp.float32)
            for t_dx in range(3):
                k = t_dy * 3 + t_dx
                qn = ex[k] * rs
                P = (ex[k][:, None] * Xn).reshape(bc * ch, w)
                cn = cn + _mm(rgbc_ref[...], _mm(P, e_ref[t_dx]))
                cd = cd + _mm(rgb_ref[...], _mm(qn.reshape(b * ch, w),
                                                e_ref[t_dx]))
            cn_cols.append(cn)
            cd_cols.append(cd)
        cnT = jnp.concatenate(cn_cols, axis=1).T
        cdT = jnp.concatenate(cd_cols, axis=1).T
        for t_dy, dy in enumerate((-1, 0, 1)):
            rp = jnp.clip(r + dy, 0, nh - 1)
            num_s[pl.ds(rp * nw, nw), :] += cnT[t_dy * nw:(t_dy + 1) * nw]
            den_s[pl.ds(rp * nw, nw), :] += cdT[t_dy * nw:(t_dy + 1) * nw]

    @pl.when(r == nh - 1)
    def _finalize():
        den = den_s[...]
        den_bc = jax.lax.dot_general(
            den, rrep_ref[...], (((1,), (0,)), ((), ())),
            preferred_element_type=jnp.float32)
        denom = jnp.where(i == 0, jnp.maximum(den_bc, 1.0), den_bc + 1e-16)
        spf = num_s[...] / denom
        spf_s[...] = spf
        num_s[...] = jnp.zeros_like(num_s)
        den_s[...] = jnp.zeros_like(den_s)

        @pl.when(i == n_iters)
        def _emit_spf():
            spfp_ref[...] = spf


@jax.jit
def kernel(x):
    b, c, h, w = x.shape
    nh, nw, ch, cw = _cells_layout(h, w, _N_SPIXELS)
    assert nh * ch == h and nw * cw == w, "kernel assumes even cell tiling"
    n_sp = nh * nw
    E, G, cbias, RGbc, RGb, R = _consts(h, w, nh, nw, ch, cw, b, c)
    grid = (_N_ITERS + 1, nh)
    body = functools.partial(_ssn_body, nh=nh, nw=nw, ch=ch, b=b, c=c,
                             n_iters=_N_ITERS)
    q, spf_p = pl.pallas_call(
        body,
        grid=grid,
        in_specs=[
            pl.BlockSpec((b, c, ch, w), lambda i, r: (0, 0, r, 0)),
            pl.BlockSpec((3, w, nw), lambda i, r: (0, 0, 0)),
            pl.BlockSpec((3, nw, w), lambda i, r: (0, 0, 0)),
            pl.BlockSpec((3, 1, w), lambda i, r: (0, 0, 0)),
            pl.BlockSpec((b * c, b * c * ch), lambda i, r: (0, 0)),
            pl.BlockSpec((b, b * ch), lambda i, r: (0, 0)),
            pl.BlockSpec((b, b * c), lambda i, r: (0, 0)),
        ],
        out_specs=[
            pl.BlockSpec((b, 9, ch, w),
                         lambda i, r: (0, 0, jnp.where(i == _N_ITERS, r, 0), 0)),
            pl.BlockSpec((n_sp, b * c), lambda i, r: (0, 0)),
        ],
        out_shape=[
            jax.ShapeDtypeStruct((b, 9, h, w), jnp.float32),
            jax.ShapeDtypeStruct((n_sp, b * c), jnp.float32),
        ],
        scratch_shapes=[
            pltpu.VMEM((n_sp, b * c), jnp.float32),
            pltpu.VMEM((n_sp, b * c), jnp.float32),
            pltpu.VMEM((n_sp, b), jnp.float32),
        ],
        compiler_params=pltpu.CompilerParams(
            dimension_semantics=("arbitrary", "arbitrary")),
    )(x, E, G, cbias, RGbc, RGb, R)
    spf_out = spf_p.T.reshape(b, c, n_sp)
    return (q, x, spf_out, x)


# speedup vs baseline: 1.5479x; 1.5479x over previous
import functools
import math

import jax
import jax.numpy as jnp
import numpy as np
from jax.experimental import pallas as pl
from jax.experimental.pallas import tpu as pltpu

_N_SPIXELS = 256
_N_ITERS = 5
_NEG = -1e16


def _cells_layout(h, w, n_spixels):
    nw = int(math.sqrt(n_spixels * w / h) + 0.5)
    nh = int(math.sqrt(n_spixels * h / w) + 0.5)
    cw = int(math.ceil(w / nw))
    ch = int(math.ceil(h / nh))
    return nh, nw, ch, cw


def _consts(h, w, nh, nw, ch, cw, b, c):
    j_of_l = np.minimum(np.arange(w) // cw, nw - 1)
    E = np.zeros((3, w, nw), np.float32)
    cbias = np.zeros((3, 1, w), np.float32)
    for t, dx in enumerate((-1, 0, 1)):
        jj = j_of_l + dx
        ok = (jj >= 0) & (jj < nw)
        jc = np.clip(jj, 0, nw - 1)
        E[t, np.arange(w), jc] = 1.0
        cbias[t, 0] = np.where(ok, 0.0, -1e16).astype(np.float32)
    G = np.ascontiguousarray(np.transpose(E, (0, 2, 1)))
    R = np.zeros((b, b * c), np.float32)
    for bi in range(b):
        R[bi, bi * c:(bi + 1) * c] = 1.0
    return (jnp.asarray(E), jnp.asarray(G), jnp.asarray(cbias),
            jnp.asarray(R))


def _scat(e_k, col):
    return jax.lax.dot_general(e_k, col, (((0,), (1,)), ((), ())),
                               preferred_element_type=jnp.float32)


def _ssn_body(x_ref, e_ref, g_ref, cbias_ref, rrep_ref,
              q_ref, spfp_ref, spf_s, num_s, den_s,
              *, nh, nw, ch, b, c, n_iters):
    i = pl.program_id(0)
    r = pl.program_id(1)
    w = x_ref.shape[-1]
    bc = b * c
    X = x_ref[...]
    X20 = X.reshape(bc, ch, w)

    @pl.when(jnp.logical_and(i == 0, r == 0))
    def _zero():
        num_s[...] = jnp.zeros_like(num_s)
        den_s[...] = jnp.zeros_like(den_s)

    @pl.when(i == 0)
    def _init():
        colsum = jnp.sum(X20, axis=1)
        cnt = jnp.full((b, w), float(ch), jnp.float32)
        num_s[pl.ds(r * nw, nw), :] += _scat(e_ref[1], colsum)
        den_s[pl.ds(r * nw, nw), :] += _scat(e_ref[1], cnt)

    @pl.when(i > 0)
    def _iterate():
        xsq = jnp.sum(X * X, axis=1)
        nd = []
        for dy in (-1, 0, 1):
            rn = r + dy
            rbias = jnp.where(jnp.logical_and(rn >= 0, rn < nh),
                              jnp.float32(0.0), jnp.float32(_NEG))
            rp = jnp.clip(rn, 0, nh - 1)
            S = spf_s[pl.ds(rp * nw, nw), :]
            for t_dx in range(3):
                Map = jax.lax.dot_general(
                    S, g_ref[t_dx], (((0,), (0,)), ((), ())),
                    preferred_element_type=jnp.float32)
                Mb = Map.reshape(b, c, 1, w)
                cross = jnp.sum(X * Mb, axis=1)
                ssq = jnp.sum(Map.reshape(b, c, w) ** 2, axis=1)
                bias = cbias_ref[t_dx] + rbias
                d = (xsq - 2.0 * cross) + ssq[:, None, :]
                nd.append(bias - d)
        m = nd[0]
        for t in range(1, 9):
            m = jnp.maximum(m, nd[t])
        ex = [jnp.exp(nd[t] - m) for t in range(9)]
        s = ex[0]
        for t in range(1, 9):
            s = s + ex[t]
        rs = 1.0 / s
        Xn = X * rs[:, None]

        @pl.when(i == n_iters)
        def _emit_q():
            q_ref[...] = jnp.stack([ex[t] * rs for t in range(9)], axis=1)

        for t_dy, dy in enumerate((-1, 0, 1)):
            rp = jnp.clip(r + dy, 0, nh - 1)
            cn = jnp.zeros((nw, bc), jnp.float32)
            cd = jnp.zeros((nw, b), jnp.float32)
            for t_dx in range(3):
                k = t_dy * 3 + t_dx
                qn = ex[k] * rs
                P = ex[k][:, None] * Xn
                colP = jnp.sum(P, axis=2).reshape(bc, w)
                colQ = jnp.sum(qn, axis=1)
                cn = cn + _scat(e_ref[t_dx], colP)
                cd = cd + _scat(e_ref[t_dx], colQ)
            num_s[pl.ds(rp * nw, nw), :] += cn
            den_s[pl.ds(rp * nw, nw), :] += cd

    @pl.when(r == nh - 1)
    def _finalize():
        den = den_s[...]
        den_bc = jax.lax.dot_general(
            den, rrep_ref[...], (((1,), (0,)), ((), ())),
            preferred_element_type=jnp.float32)
        denom = jnp.where(i == 0, jnp.maximum(den_bc, 1.0), den_bc + 1e-16)
        spf = num_s[...] / denom
        spf_s[...] = spf
        num_s[...] = jnp.zeros_like(num_s)
        den_s[...] = jnp.zeros_like(den_s)

        @pl.when(i == n_iters)
        def _emit_spf():
            spfp_ref[...] = spf


@jax.jit
def kernel(x):
    b, c, h, w = x.shape
    nh, nw, ch, cw = _cells_layout(h, w, _N_SPIXELS)
    assert nh * ch == h and nw * cw == w, "kernel assumes even cell tiling"
    n_sp = nh * nw
    E, G, cbias, R = _consts(h, w, nh, nw, ch, cw, b, c)
    grid = (_N_ITERS + 1, nh)
    body = functools.partial(_ssn_body, nh=nh, nw=nw, ch=ch, b=b, c=c,
                             n_iters=_N_ITERS)
    q, spf_p = pl.pallas_call(
        body,
        grid=grid,
        in_specs=[
            pl.BlockSpec((b, c, ch, w), lambda i, r: (0, 0, r, 0)),
            pl.BlockSpec((3, w, nw), lambda i, r: (0, 0, 0)),
            pl.BlockSpec((3, nw, w), lambda i, r: (0, 0, 0)),
            pl.BlockSpec((3, 1, w), lambda i, r: (0, 0, 0)),
            pl.BlockSpec((b, b * c), lambda i, r: (0, 0)),
        ],
        out_specs=[
            pl.BlockSpec((b, 9, ch, w),
                         lambda i, r: (0, 0, jnp.where(i == _N_ITERS, r, 0), 0)),
            pl.BlockSpec((n_sp, b * c), lambda i, r: (0, 0)),
        ],
        out_shape=[
            jax.ShapeDtypeStruct((b, 9, h, w), jnp.float32),
            jax.ShapeDtypeStruct((n_sp, b * c), jnp.float32),
        ],
        scratch_shapes=[
            pltpu.VMEM((n_sp, b * c), jnp.float32),
            pltpu.VMEM((n_sp, b * c), jnp.float32),
            pltpu.VMEM((n_sp, b), jnp.float32),
        ],
        compiler_params=pltpu.CompilerParams(
            dimension_semantics=("arbitrary", "arbitrary")),
    )(x, E, G, cbias, R)
    spf_out = spf_p.T.reshape(b, c, n_sp)
    return (q, x, spf_out, x)
